# Initial kernel scaffold; baseline (speedup 1.0000x reference)
import jax
import jax.numpy as jnp
from jax.experimental import pallas as pl


def kernel(x, coeffs, knots):
    raise NotImplementedError("write your pallas kernel here")


# speedup vs baseline: 924.4710x; 924.4710x over previous
import functools

import jax
import jax.numpy as jnp
from jax import lax
from jax.experimental import pallas as pl
from jax.experimental.pallas import tpu as pltpu
from jax.experimental.pallas import tpu_sc as plsc

L = 16
NC = 2
NS = 16
NW = NC * NS

ROWS, COLS = 4096, 8192
TOTAL = ROWS * COLS
PER_W = TOTAL // NW
CHUNK = 16384
NCHUNK = PER_W // CHUNK
NK = 1024


def _body(x_hbm, coeffs_hbm, k0_hbm, invh_hbm, out_hbm,
          coeffs_v, k0_v, invh_v, xbuf, obuf):
    wid = lax.axis_index("s") * NC + lax.axis_index("c")
    base = wid * PER_W

    pltpu.sync_copy(coeffs_hbm, coeffs_v)
    pltpu.sync_copy(k0_hbm, k0_v)
    pltpu.sync_copy(invh_hbm, invh_v)

    k0 = k0_v[...]
    invh = invh_v[...]

    def chunk_body(ci, carry):
        off = base + ci * CHUNK
        pltpu.sync_copy(x_hbm.at[pl.ds(off, CHUNK)], xbuf)

        @plsc.parallel_loop(0, CHUNK, step=L, unroll=8)
        def _vec(j):
            xv = xbuf[pl.ds(j, L)]
            g = (xv - k0) * invh
            i = jnp.clip(g.astype(jnp.int32), 0, NK - 2)
            t = g - i.astype(jnp.float32)
            c0 = plsc.load_gather(coeffs_v, [i])
            c1 = plsc.load_gather(coeffs_v, [i + 1])
            obuf[pl.ds(j, L)] = c0 + t * (c1 - c0)

        pltpu.sync_copy(obuf, out_hbm.at[pl.ds(off, CHUNK)])
        return carry

    lax.fori_loop(0, NCHUNK, chunk_body, 0)


@functools.partial(
    pl.kernel,
    out_type=jax.ShapeDtypeStruct((TOTAL,), jnp.float32),
    mesh=plsc.VectorSubcoreMesh(
        core_axis_name="c", subcore_axis_name="s",
        num_cores=NC, num_subcores=NS),
    compiler_params=pltpu.CompilerParams(needs_layout_passes=False),
    scratch_types=[
        pltpu.VMEM((NK,), jnp.float32),
        pltpu.VMEM((L,), jnp.float32),
        pltpu.VMEM((L,), jnp.float32),
        pltpu.VMEM((CHUNK,), jnp.float32),
        pltpu.VMEM((CHUNK,), jnp.float32),
    ],
)
def _spline_sc(x_hbm, coeffs_hbm, k0_hbm, invh_hbm, out_hbm,
               coeffs_v, k0_v, invh_v, xbuf, obuf):
    _body(x_hbm, coeffs_hbm, k0_hbm, invh_hbm, out_hbm,
          coeffs_v, k0_v, invh_v, xbuf, obuf)


def kernel(x, coeffs, knots):
    k0 = knots[0]
    invh = (NK - 1) / (knots[-1] - k0)
    k0s = jnp.full((L,), k0, jnp.float32)
    invhs = jnp.full((L,), invh, jnp.float32)
    out = _spline_sc(x.reshape(TOTAL), coeffs, k0s, invhs)
    return out.reshape(x.shape)


# speedup vs baseline: 1191.7185x; 1.2891x over previous
import functools

import jax
import jax.numpy as jnp
from jax import lax
from jax.experimental import pallas as pl
from jax.experimental.pallas import tpu as pltpu
from jax.experimental.pallas import tpu_sc as plsc

L = 16
NC = 2
NS = 16
NW = NC * NS

ROWS, COLS = 4096, 8192
TOTAL = ROWS * COLS
PER_W = TOTAL // NW
CHUNK = 16384
NCHUNK = PER_W // CHUNK
NPAIR = NCHUNK // 2
NK = 1024
UNROLL = 8


def _body(x_hbm, coeffs_hbm, k0_hbm, invh_hbm, out_hbm,
          coeffs_v, k0_v, invh_v, xbuf_a, xbuf_b, obuf_a, obuf_b,
          insem_a, insem_b, outsem_a, outsem_b):
    wid = lax.axis_index("s") * NC + lax.axis_index("c")
    base = wid * PER_W

    pltpu.sync_copy(coeffs_hbm, coeffs_v)
    pltpu.sync_copy(k0_hbm, k0_v)
    pltpu.sync_copy(invh_hbm, invh_v)

    k0 = k0_v[...]
    invh = invh_v[...]

    def compute(xbuf, obuf):
        @plsc.parallel_loop(0, CHUNK, step=L, unroll=UNROLL)
        def _vec(j):
            xv = xbuf[pl.ds(j, L)]
            g = (xv - k0) * invh
            i = jnp.clip(g.astype(jnp.int32), 0, NK - 2)
            t = g - i.astype(jnp.float32)
            c0 = plsc.load_gather(coeffs_v, [i])
            c1 = plsc.load_gather(coeffs_v, [i + 1])
            obuf[pl.ds(j, L)] = c0 + t * (c1 - c0)

    def in_slice(ci):
        return x_hbm.at[pl.ds(base + ci * CHUNK, CHUNK)]

    def out_slice(ci):
        return out_hbm.at[pl.ds(base + ci * CHUNK, CHUNK)]

    pltpu.async_copy(in_slice(0), xbuf_a, insem_a)

    def pair(k, carry):
        ci0 = 2 * k
        ci1 = 2 * k + 1

        pltpu.async_copy(in_slice(ci1), xbuf_b, insem_b)
        pltpu.make_async_copy(in_slice(ci0), xbuf_a, insem_a).wait()

        @pl.when(k > 0)
        def _():
            pltpu.make_async_copy(obuf_a, out_slice(ci0), outsem_a).wait()

        compute(xbuf_a, obuf_a)
        pltpu.async_copy(obuf_a, out_slice(ci0), outsem_a)

        @pl.when(k + 1 < NPAIR)
        def _():
            pltpu.async_copy(in_slice(ci0 + 2), xbuf_a, insem_a)

        pltpu.make_async_copy(in_slice(ci1), xbuf_b, insem_b).wait()

        @pl.when(k > 0)
        def _():
            pltpu.make_async_copy(obuf_b, out_slice(ci1), outsem_b).wait()

        compute(xbuf_b, obuf_b)
        pltpu.async_copy(obuf_b, out_slice(ci1), outsem_b)
        return carry

    lax.fori_loop(0, NPAIR, pair, 0)

    pltpu.make_async_copy(obuf_a, out_slice(NCHUNK - 2), outsem_a).wait()
    pltpu.make_async_copy(obuf_b, out_slice(NCHUNK - 1), outsem_b).wait()


@functools.partial(
    pl.kernel,
    out_type=jax.ShapeDtypeStruct((TOTAL,), jnp.float32),
    mesh=plsc.VectorSubcoreMesh(
        core_axis_name="c", subcore_axis_name="s",
        num_cores=NC, num_subcores=NS),
    compiler_params=pltpu.CompilerParams(needs_layout_passes=False),
    scratch_types=[
        pltpu.VMEM((NK,), jnp.float32),
        pltpu.VMEM((L,), jnp.float32),
        pltpu.VMEM((L,), jnp.float32),
        pltpu.VMEM((CHUNK,), jnp.float32),
        pltpu.VMEM((CHUNK,), jnp.float32),
        pltpu.VMEM((CHUNK,), jnp.float32),
        pltpu.VMEM((CHUNK,), jnp.float32),
        pltpu.SemaphoreType.DMA,
        pltpu.SemaphoreType.DMA,
        pltpu.SemaphoreType.DMA,
        pltpu.SemaphoreType.DMA,
    ],
)
def _spline_sc(*refs):
    _body(*refs)


def kernel(x, coeffs, knots):
    k0 = knots[0]
    invh = (NK - 1) / (knots[-1] - k0)
    k0s = jnp.full((L,), k0, jnp.float32)
    invhs = jnp.full((L,), invh, jnp.float32)
    out = _spline_sc(x.reshape(TOTAL), coeffs, k0s, invhs)
    return out.reshape(x.shape)
